# baseline (speedup 1.0000x reference)
import functools

import jax
import jax.numpy as jnp
from jax.experimental import pallas as pl
from jax.experimental.pallas import tpu as pltpu

B = 256
N = 64
ENC = 128
DEC = 64
DEC2 = 32
CB = 8
STEPS = N // CB
K1 = N * DEC
P1 = N * DEC2
WB = CB * DEC2


def _goggle_kernel(x_ref, we_ref, be_ref, wmu_ref, bmu_ref, wlv_ref, blv_ref,
                   g_ref, gt_ref, w0f_ref, ccf_ref, w1_ref, b1e_ref, w2m_ref,
                   b2_ref, it_ref, eps_ref,
                   xhat_ref, adj_ref, mu_ref, lv_ref,
                   bz_ref, h1_ref, adjt_ref):
    k = pl.program_id(0)

    @pl.when(k == 0)
    def _prologue():
        h = jax.nn.relu(jnp.dot(x_ref[...], we_ref[...],
                                preferred_element_type=jnp.float32) + be_ref[...])
        mu = jnp.dot(h, wmu_ref[...], preferred_element_type=jnp.float32) + bmu_ref[...]
        lv = jnp.dot(h, wlv_ref[...], preferred_element_type=jnp.float32) + blv_ref[...]
        mu_ref[...] = mu
        lv_ref[...] = lv
        z = mu + eps_ref[...] * jnp.exp(0.5 * lv)

        r_id = jax.lax.broadcasted_iota(jnp.int32, (N, N), 0)
        c_id = jax.lax.broadcasted_iota(jnp.int32, (N, N), 1)
        eye = (r_id == c_id).astype(jnp.float32)
        thr = it_ref[0, 0] > 50.0
        adj = jax.nn.sigmoid(g_ref[...]) * (1.0 - eye) + eye
        adj = jnp.where(jnp.logical_and(thr, adj <= 0.1), 0.0, adj)
        adj_ref[...] = adj
        adjt = jax.nn.sigmoid(gt_ref[...]) * (1.0 - eye) + eye
        adjt_ref[...] = jnp.where(jnp.logical_and(thr, adjt <= 0.1), 0.0, adjt)

        ez = (jax.lax.broadcasted_iota(jnp.int32, (N, K1), 1) // DEC
              == jax.lax.broadcasted_iota(jnp.int32, (N, K1), 0)).astype(jnp.float32)
        zexp = jnp.dot(z, ez, preferred_element_type=jnp.float32)
        bz_ref[...] = jnp.tanh(zexp * w0f_ref[...] + ccf_ref[...])

    adjcols = adjt_ref[pl.ds(k * CB, CB), :]
    er = (jax.lax.broadcasted_iota(jnp.int32, (K1, N), 0) // DEC
          == jax.lax.broadcasted_iota(jnp.int32, (K1, N), 1)).astype(jnp.float32)
    sblk = jax.lax.dot_general(er, adjcols, (((1,), (1,)), ((), ())),
                               preferred_element_type=jnp.float32)
    rexp = (jax.lax.broadcasted_iota(jnp.int32, (CB, WB), 1) // DEC2
            == jax.lax.broadcasted_iota(jnp.int32, (CB, WB), 0)).astype(jnp.float32)
    scale = jnp.dot(sblk, rexp, preferred_element_type=jnp.float32)
    w2d = jnp.concatenate([w1_ref[:, j].reshape(K1, DEC2) for j in range(CB)],
                          axis=1)
    res = jnp.dot(bz_ref[...], w2d * scale, preferred_element_type=jnp.float32)
    h1_ref[:, pl.ds(k * WB, WB)] = jax.nn.relu(
        res + b1e_ref[:, pl.ds(k * WB, WB)])

    @pl.when(k == STEPS - 1)
    def _epilogue():
        er2 = (jax.lax.broadcasted_iota(jnp.int32, (P1, N), 0) // DEC2
               == jax.lax.broadcasted_iota(jnp.int32, (P1, N), 1)).astype(jnp.float32)
        s2 = jnp.dot(er2, adj_ref[...], preferred_element_type=jnp.float32)
        xhat_ref[...] = jnp.dot(h1_ref[...], w2m_ref[...] * s2,
                                preferred_element_type=jnp.float32) + b2_ref[...]


@functools.partial(jax.jit, static_argnames=())
def kernel(x, We, be, Wmu, bmu, Wlv, blv, G, Wemb, bemb, W1, b1, W2, b2, iter):
    f32 = jnp.float32
    w0f = Wemb[:, 0, :].reshape(1, K1)
    ccf = (Wemb[jnp.arange(N), jnp.arange(N) + 1, :] + bemb).reshape(1, K1)
    w2m = W2[:, :, :, 0].transpose(0, 2, 1).reshape(P1, N)
    b1e = jnp.tile(b1, N).reshape(1, P1)
    eps = jax.random.normal(jax.random.key(42), (B, N), dtype=f32)
    it = jnp.asarray(iter, dtype=f32).reshape(1, 1)

    resident = lambda s: pl.BlockSpec(s, lambda k: (0,) * len(s))
    out = pl.pallas_call(
        _goggle_kernel,
        grid=(STEPS,),
        in_specs=[
            resident((B, N)),
            resident((N, ENC)),
            resident((1, ENC)),
            resident((ENC, N)),
            resident((1, N)),
            resident((ENC, N)),
            resident((1, N)),
            resident((N, N)),
            resident((N, N)),
            resident((1, K1)),
            resident((1, K1)),
            pl.BlockSpec((N, CB, DEC, DEC2), lambda k: (0, k, 0, 0)),
            resident((1, P1)),
            resident((P1, N)),
            resident((1, 1)),
            resident((1, 1)),
            resident((B, N)),
        ],
        out_specs=(
            resident((B, N)),
            resident((N, N)),
            resident((B, N)),
            resident((B, N)),
        ),
        out_shape=(
            jax.ShapeDtypeStruct((B, N), f32),
            jax.ShapeDtypeStruct((N, N), f32),
            jax.ShapeDtypeStruct((B, N), f32),
            jax.ShapeDtypeStruct((B, N), f32),
        ),
        scratch_shapes=[
            pltpu.VMEM((B, K1), f32),
            pltpu.VMEM((B, P1), f32),
            pltpu.VMEM((N, N), f32),
        ],
        compiler_params=pltpu.CompilerParams(
            dimension_semantics=("arbitrary",),
        ),
    )(x, We, be.reshape(1, ENC), Wmu, bmu.reshape(1, N), Wlv, blv.reshape(1, N),
      G, G.T, w0f, ccf, W1, b1e, w2m, b2.reshape(1, 1), it, eps)
    return out
